# baseline (speedup 1.0000x reference)
import dataclasses
import functools

import jax
import jax.numpy as jnp
from jax.experimental import pallas as pl
from jax.experimental.pallas import tpu as pltpu
from jax.experimental.pallas import tpu_sc as plsc

_VOCAB = 26
_DIM = 64
_LANES = 16
_CHUNK = 512


def _mod26(v):
    s = (v >= 26).astype(jnp.int32)
    s += (v >= 52).astype(jnp.int32)
    s += (v >= 78).astype(jnp.int32)
    s += (v >= 104).astype(jnp.int32)
    return v - 26 * s


def kernel(seq, W):
    n = seq.shape[0]
    wt = W.T.reshape(-1)

    mesh = plsc.VectorSubcoreMesh(core_axis_name="c", subcore_axis_name="s")
    cp = pltpu.CompilerParams()
    if "needs_layout_passes" in pltpu.CompilerParams.__dataclass_fields__:
        cp = dataclasses.replace(cp, needs_layout_passes=False)

    @functools.partial(
        pl.kernel,
        out_type=jax.ShapeDtypeStruct((_DIM, n), W.dtype),
        mesh=mesh,
        scratch_types=[
            pltpu.VMEM((_VOCAB * _DIM,), jnp.float32),
            pltpu.SemaphoreType.DMA,
        ],
        compiler_params=cp,
    )
    def emb(seq_hbm, wt_hbm, out_hbm, wt_v, sem):
        pltpu.async_copy(wt_hbm, wt_v, sem).wait()

        def body(seq_vmem, out_vmem):
            @plsc.parallel_loop(0, _CHUNK, step=_LANES, unroll=4)
            def _(c):
                sl = pl.ds(c, _LANES)
                v = _mod26(seq_vmem[sl])
                for d in range(_DIM):
                    out_vmem[d, sl] = plsc.load_gather(wt_v, [v + d * _VOCAB])

        pltpu.emit_pipeline(
            body,
            grid=(n // _CHUNK,),
            in_specs=[pl.BlockSpec((_CHUNK,), lambda i: (i,))],
            out_specs=[pl.BlockSpec((_DIM, _CHUNK), lambda i: (0, i))],
            core_axis_name=("c", "s"),
            dimension_semantics=(pltpu.PARALLEL,),
        )(seq_hbm, out_hbm)

    return emb(seq, wt).T


# speedup vs baseline: 1.4040x; 1.4040x over previous
import dataclasses
import functools

import jax
import jax.numpy as jnp
from jax.experimental import pallas as pl
from jax.experimental.pallas import tpu as pltpu
from jax.experimental.pallas import tpu_sc as plsc

_VOCAB = 26
_DIM = 64
_LANES = 16
_CHUNK = 640


def _mod26(v):
    s = (v >= 26).astype(jnp.int32)
    s += (v >= 52).astype(jnp.int32)
    s += (v >= 78).astype(jnp.int32)
    s += (v >= 104).astype(jnp.int32)
    return v - 26 * s


def kernel(seq, W):
    n = seq.shape[0]
    wt = W.T.reshape(-1)

    mesh = plsc.VectorSubcoreMesh(core_axis_name="c", subcore_axis_name="s")
    cp = pltpu.CompilerParams()
    if "needs_layout_passes" in pltpu.CompilerParams.__dataclass_fields__:
        cp = dataclasses.replace(cp, needs_layout_passes=False)

    @functools.partial(
        pl.kernel,
        out_type=jax.ShapeDtypeStruct((_DIM, n), W.dtype),
        mesh=mesh,
        scratch_types=[
            pltpu.VMEM((_VOCAB * _DIM,), jnp.float32),
            pltpu.SemaphoreType.DMA,
        ],
        compiler_params=cp,
    )
    def emb(seq_hbm, wt_hbm, out_hbm, wt_v, sem):
        pltpu.async_copy(wt_hbm, wt_v, sem).wait()

        def body(seq_vmem, out_vmem):
            @plsc.parallel_loop(0, _CHUNK, step=_LANES, unroll=2)
            def _(c):
                sl = pl.ds(c, _LANES)
                v = _mod26(seq_vmem[sl])
                for d in range(_DIM):
                    out_vmem[d, sl] = plsc.load_gather(wt_v, [v + d * _VOCAB])

        pltpu.emit_pipeline(
            body,
            grid=(n // _CHUNK,),
            in_specs=[pl.BlockSpec((_CHUNK,), lambda i: (i,))],
            out_specs=[pl.BlockSpec((_DIM, _CHUNK), lambda i: (0, i))],
            core_axis_name=("c", "s"),
            dimension_semantics=(pltpu.PARALLEL,),
        )(seq_hbm, out_hbm)

    return emb(seq, wt).T
